# baseline (speedup 1.0000x reference)
import functools

import jax
import jax.numpy as jnp
from jax import lax
from jax.experimental import pallas as pl
from jax.experimental.pallas import tpu as pltpu
from jax.experimental.pallas import tpu_sc as plsc

_NUM_BINS = 256
_NBP = 264
_LO = 0.0
_HI = 255.0
_NW = 32
_LANES = 16
_CHUNK = 8192
_HSIZE = _LANES * _NBP


_NBUF = 12


def _sc_hist_body(n_elems, chunk, in_hbm, tgt_hbm, out_hbm, *refs):
    bufs = refs[:_NBUF]
    hist = refs[_NBUF]
    sems = refs[_NBUF + 1:]
    cid = lax.axis_index("c")
    sid = lax.axis_index("s")
    wid = sid * 2 + cid
    per_worker = n_elems // _NW
    n_chunks = per_worker // chunk
    inv_w = jnp.float32(_NUM_BINS / (_HI - _LO))
    lane_base = lax.iota(jnp.int32, _LANES) * _NBP
    ones = jnp.ones((_LANES,), jnp.float32)

    steps = []
    for img, src in ((0, in_hbm), (1, tgt_hbm)):
        for c in range(n_chunks):
            steps.append((src, c, img * _HSIZE))

    n_cols = in_hbm.shape[1]
    rows_per_chunk = chunk // n_cols

    def start(s, b):
        src, c, _ = steps[s]
        row0 = pl.multiple_of((wid * per_worker + c * chunk) // n_cols, 8)
        return pltpu.async_copy(
            src.at[pl.ds(row0, rows_per_chunk), :], bufs[b], sems[b])

    copies = [None] * _NBUF
    for p in range(_NBUF - 1):
        copies[p] = start(p, p)

    def zbody(i, _):
        hist[pl.ds(i * _LANES, _LANES)] = jnp.zeros((_LANES,), jnp.float32)
        return 0
    lax.fori_loop(0, (2 * _HSIZE) // _LANES, zbody, 0)

    for s in range(len(steps)):
        b = s % _NBUF
        if s + _NBUF - 1 < len(steps):
            copies[(s + _NBUF - 1) % _NBUF] = start(s + _NBUF - 1,
                                                    (s + _NBUF - 1) % _NBUF)
        copies[b].wait()
        buf = bufs[b]
        base_vec = lane_base + steps[s][2]
        vecs_per_row = n_cols // _LANES

        @functools.partial(plsc.parallel_loop, 0, chunk // _LANES, unroll=16)
        def body(i, buf=buf, base_vec=base_vec):
            r = i // vecs_per_row
            c = (i % vecs_per_row) * _LANES
            x = buf[r, pl.ds(c, _LANES)]
            idx = (x * inv_w).astype(jnp.int32)
            plsc.addupdate_scatter(hist, (idx + base_vec,), ones)

    pltpu.sync_copy(hist.at[pl.ds(0, _HSIZE)], out_hbm.at[wid])
    pltpu.sync_copy(hist.at[pl.ds(_HSIZE, _HSIZE)], out_hbm.at[_NW + wid])


def _tc_loss_body(p_ref, o_ref):
    p = p_ref[...]
    half = p.shape[0] // 2
    h0 = jnp.sum(p[:half], axis=0, keepdims=True)
    h1 = jnp.sum(p[half:], axis=0, keepdims=True)
    col = lax.broadcasted_iota(jnp.int32, h0.shape, 1)
    spill0 = jnp.sum(jnp.where(col == _NUM_BINS, h0, 0.0))
    spill1 = jnp.sum(jnp.where(col == _NUM_BINS, h1, 0.0))
    h0 = h0 + jnp.where(col == _NUM_BINS - 1, spill0, 0.0)
    h1 = h1 + jnp.where(col == _NUM_BINS - 1, spill1, 0.0)
    s0 = jnp.sum(h0) - spill0
    s1 = jnp.sum(h1) - spill1
    m = jnp.minimum(h0 / s0, h1 / s1) * (col < _NUM_BINS)
    loss = 1.0 - jnp.sum(m)
    o_ref[...] = jnp.full((8, 128), loss, jnp.float32)


def kernel(input_image, target_image):
    n = input_image.size
    n_cols = input_image.shape[-1]
    x = input_image.reshape(-1, n_cols)
    t = target_image.reshape(-1, n_cols)

    chunk = _CHUNK
    per_worker = n // _NW
    while per_worker % chunk != 0 or chunk % n_cols != 0:
        chunk //= 2

    mesh = plsc.VectorSubcoreMesh(core_axis_name="c", subcore_axis_name="s")
    sc = pl.kernel(
        functools.partial(_sc_hist_body, n, chunk),
        out_type=jax.ShapeDtypeStruct((2 * _NW, _HSIZE), jnp.float32),
        mesh=mesh,
        scratch_types=(
            [pltpu.VMEM((chunk // n_cols, n_cols), jnp.float32)] * _NBUF
            + [pltpu.VMEM((2 * _HSIZE,), jnp.float32)]
            + [pltpu.SemaphoreType.DMA] * _NBUF
        ),
        compiler_params=pltpu.CompilerParams(needs_layout_passes=False),
    )
    partials = sc(x, t)
    partials = partials.reshape(2 * _NW * _LANES, _NBP)

    loss_tile = pl.pallas_call(
        _tc_loss_body,
        out_shape=jax.ShapeDtypeStruct((8, 128), jnp.float32),
    )(partials)
    return loss_tile[0, 0]


# speedup vs baseline: 1.0661x; 1.0661x over previous
import functools

import jax
import jax.numpy as jnp
from jax import lax
from jax.experimental import pallas as pl
from jax.experimental.pallas import tpu as pltpu
from jax.experimental.pallas import tpu_sc as plsc

_NUM_BINS = 256
_NBP = 264
_LO = 0.0
_HI = 255.0
_NW = 32
_LANES = 16
_CHUNK = 16384
_HSIZE = _LANES * _NBP


_NBUF = 7


def _sc_hist_body(n_elems, chunk, in_hbm, tgt_hbm, out_hbm, *refs):
    bufs = refs[:_NBUF]
    hist = refs[_NBUF]
    sems = refs[_NBUF + 1:]
    cid = lax.axis_index("c")
    sid = lax.axis_index("s")
    wid = sid * 2 + cid
    per_worker = n_elems // _NW
    n_chunks = per_worker // chunk
    inv_w = jnp.float32(_NUM_BINS / (_HI - _LO))
    lane_base = lax.iota(jnp.int32, _LANES) * _NBP
    ones = jnp.ones((_LANES,), jnp.float32)

    steps = []
    for img, src in ((0, in_hbm), (1, tgt_hbm)):
        for c in range(n_chunks):
            steps.append((src, c, img * _HSIZE))

    n_cols = in_hbm.shape[1]
    rows_per_chunk = chunk // n_cols

    def start(s, b):
        src, c, _ = steps[s]
        row0 = pl.multiple_of((wid * per_worker + c * chunk) // n_cols, 8)
        return pltpu.async_copy(
            src.at[pl.ds(row0, rows_per_chunk), :], bufs[b], sems[b])

    copies = [None] * _NBUF
    for p in range(_NBUF - 1):
        copies[p] = start(p, p)

    def zbody(i, _):
        hist[pl.ds(i * _LANES, _LANES)] = jnp.zeros((_LANES,), jnp.float32)
        return 0
    lax.fori_loop(0, (2 * _HSIZE) // _LANES, zbody, 0)

    for s in range(len(steps)):
        b = s % _NBUF
        if s + _NBUF - 1 < len(steps):
            copies[(s + _NBUF - 1) % _NBUF] = start(s + _NBUF - 1,
                                                    (s + _NBUF - 1) % _NBUF)
        copies[b].wait()
        buf = bufs[b]
        base_vec = lane_base + steps[s][2]
        vecs_per_row = n_cols // _LANES

        @functools.partial(plsc.parallel_loop, 0, chunk // _LANES, unroll=16)
        def body(i, buf=buf, base_vec=base_vec):
            r = i // vecs_per_row
            c = (i % vecs_per_row) * _LANES
            x = buf[r, pl.ds(c, _LANES)]
            idx = (x * inv_w).astype(jnp.int32)
            plsc.addupdate_scatter(hist, (idx + base_vec,), ones)

    pltpu.sync_copy(hist.at[pl.ds(0, _HSIZE)], out_hbm.at[wid])
    pltpu.sync_copy(hist.at[pl.ds(_HSIZE, _HSIZE)], out_hbm.at[_NW + wid])


def _tc_loss_body(p_ref, o_ref):
    p = p_ref[...]
    half = p.shape[0] // 2
    h0 = jnp.sum(p[:half], axis=0, keepdims=True)
    h1 = jnp.sum(p[half:], axis=0, keepdims=True)
    col = lax.broadcasted_iota(jnp.int32, h0.shape, 1)
    spill0 = jnp.sum(jnp.where(col == _NUM_BINS, h0, 0.0))
    spill1 = jnp.sum(jnp.where(col == _NUM_BINS, h1, 0.0))
    h0 = h0 + jnp.where(col == _NUM_BINS - 1, spill0, 0.0)
    h1 = h1 + jnp.where(col == _NUM_BINS - 1, spill1, 0.0)
    s0 = jnp.sum(h0) - spill0
    s1 = jnp.sum(h1) - spill1
    m = jnp.minimum(h0 / s0, h1 / s1) * (col < _NUM_BINS)
    loss = 1.0 - jnp.sum(m)
    o_ref[...] = jnp.full((8, 128), loss, jnp.float32)


def kernel(input_image, target_image):
    n = input_image.size
    n_cols = input_image.shape[-1]
    x = input_image.reshape(-1, n_cols)
    t = target_image.reshape(-1, n_cols)

    chunk = _CHUNK
    per_worker = n // _NW
    while per_worker % chunk != 0 or chunk % n_cols != 0:
        chunk //= 2

    mesh = plsc.VectorSubcoreMesh(core_axis_name="c", subcore_axis_name="s")
    sc = pl.kernel(
        functools.partial(_sc_hist_body, n, chunk),
        out_type=jax.ShapeDtypeStruct((2 * _NW, _HSIZE), jnp.float32),
        mesh=mesh,
        scratch_types=(
            [pltpu.VMEM((chunk // n_cols, n_cols), jnp.float32)] * _NBUF
            + [pltpu.VMEM((2 * _HSIZE,), jnp.float32)]
            + [pltpu.SemaphoreType.DMA] * _NBUF
        ),
        compiler_params=pltpu.CompilerParams(needs_layout_passes=False),
    )
    partials = sc(x, t)
    partials = partials.reshape(2 * _NW * _LANES, _NBP)

    loss_tile = pl.pallas_call(
        _tc_loss_body,
        out_shape=jax.ShapeDtypeStruct((8, 128), jnp.float32),
    )(partials)
    return loss_tile[0, 0]
